# baseline (speedup 1.0000x reference)
import functools

import jax
import jax.numpy as jnp
from jax import lax
from jax.experimental import pallas as pl
from jax.experimental.pallas import tpu as pltpu
from jax.experimental.pallas import tpu_sc as plsc

_LANES = 16
_NC = 2
_NS = 16
_NW = _NC * _NS
_CHUNK = 64


def _make_sc_reverse(maxlen, dim, dtype):
    rows_per_w = maxlen // _NW
    n_chunks = rows_per_w // _CHUNK
    lanes_per_row = dim // _LANES
    mesh = plsc.VectorSubcoreMesh(core_axis_name="c", subcore_axis_name="s")

    @functools.partial(
        pl.kernel,
        mesh=mesh,
        out_type=jax.ShapeDtypeStruct((maxlen, dim), dtype),
        scratch_types=(
            [pltpu.VMEM((_CHUNK, dim), dtype) for _ in range(2 * n_chunks)]
            + [pltpu.SemaphoreType.DMA for _ in range(2 * n_chunks)]
        ),
    )
    def rev(table_hbm, out_hbm, *scratch):
        in_bufs = scratch[:n_chunks]
        out_bufs = scratch[n_chunks:2 * n_chunks]
        in_sems = scratch[2 * n_chunks:3 * n_chunks]
        out_sems = scratch[3 * n_chunks:]
        wid = lax.axis_index("s") * _NC + lax.axis_index("c")
        base = wid * rows_per_w

        in_copies = []
        for c in range(n_chunks):
            src = maxlen - base - (c + 1) * _CHUNK
            in_copies.append(
                pltpu.async_copy(table_hbm.at[pl.ds(src, _CHUNK)],
                                 in_bufs[c], in_sems[c]))

        out_copies = []
        for c in range(n_chunks):
            in_copies[c].wait()
            for j in range(_CHUNK):
                for k in range(lanes_per_row):
                    out_bufs[c][j, pl.ds(k * _LANES, _LANES)] = (
                        in_bufs[c][_CHUNK - 1 - j, pl.ds(k * _LANES, _LANES)])
            out_copies.append(
                pltpu.async_copy(out_bufs[c],
                                 out_hbm.at[pl.ds(base + c * _CHUNK, _CHUNK)],
                                 out_sems[c]))
        for cp in out_copies:
            cp.wait()

    return rev


def kernel(x, pos_emb):
    maxlen = x.shape[-1]
    dim = pos_emb.shape[1]
    rev = _make_sc_reverse(maxlen, dim, pos_emb.dtype)
    return rev(pos_emb[:maxlen])


# speedup vs baseline: 4.3590x; 4.3590x over previous
import jax
import jax.numpy as jnp
from jax.experimental import pallas as pl
from jax.experimental.pallas import tpu as pltpu

_ROWS = 512
_DOT = 128
_NBUF = 4


def _body(in_hbm, out_hbm, *scratch):
    maxlen, dim = in_hbm.shape
    n_chunks = maxlen // _ROWS
    in_bufs = scratch[:_NBUF]
    out_bufs = scratch[_NBUF:2 * _NBUF]
    in_sems = scratch[2 * _NBUF]
    out_sems = scratch[2 * _NBUF + 1]

    rows = jax.lax.broadcasted_iota(jnp.int32, (_DOT, _DOT), 0)
    cols = jax.lax.broadcasted_iota(jnp.int32, (_DOT, _DOT), 1)
    perm = (rows + cols == _DOT - 1).astype(jnp.float32)

    def start_in(c):
        src = maxlen - (c + 1) * _ROWS
        pltpu.make_async_copy(
            in_hbm.at[pl.ds(src, _ROWS)], in_bufs[c % _NBUF],
            in_sems.at[c % _NBUF]).start()

    for c in range(min(_NBUF, n_chunks)):
        start_in(c)

    for c in range(n_chunks):
        b = c % _NBUF
        pltpu.make_async_copy(
            in_hbm.at[pl.ds(maxlen - (c + 1) * _ROWS, _ROWS)], in_bufs[b],
            in_sems.at[b]).wait()
        if c >= _NBUF:
            pltpu.make_async_copy(
                out_bufs[b], out_hbm.at[pl.ds((c - _NBUF) * _ROWS, _ROWS)],
                out_sems.at[b]).wait()
        nd = _ROWS // _DOT
        for k in range(nd):
            src = (nd - 1 - k) * _DOT
            out_bufs[b][k * _DOT:(k + 1) * _DOT, :] = jnp.dot(
                perm, in_bufs[b][src:src + _DOT, :],
                preferred_element_type=jnp.float32)
        pltpu.make_async_copy(
            out_bufs[b], out_hbm.at[pl.ds(c * _ROWS, _ROWS)],
            out_sems.at[b]).start()
        nxt = c + _NBUF
        if nxt < n_chunks:
            start_in(nxt)

    for c in range(max(0, n_chunks - _NBUF), n_chunks):
        b = c % _NBUF
        pltpu.make_async_copy(
            out_bufs[b], out_hbm.at[pl.ds(c * _ROWS, _ROWS)],
            out_sems.at[b]).wait()


def kernel(x, pos_emb):
    maxlen = x.shape[-1]
    dim = pos_emb.shape[1]
    return pl.pallas_call(
        _body,
        in_specs=[pl.BlockSpec(memory_space=pl.ANY)],
        out_specs=pl.BlockSpec(memory_space=pl.ANY),
        out_shape=jax.ShapeDtypeStruct((maxlen, dim), pos_emb.dtype),
        scratch_shapes=(
            [pltpu.VMEM((_ROWS, dim), jnp.float32) for _ in range(2 * _NBUF)]
            + [pltpu.SemaphoreType.DMA((_NBUF,)),
               pltpu.SemaphoreType.DMA((_NBUF,))]
        ),
    )(pos_emb[:maxlen])


# speedup vs baseline: 5.7306x; 1.3147x over previous
import jax
import jax.numpy as jnp
from jax.experimental import pallas as pl
from jax.experimental.pallas import tpu as pltpu

_ROWS = 1024
_DOT = 128
_NBUF = 4


def _body(in_hbm, out_hbm, *scratch):
    maxlen, dim = in_hbm.shape
    n_chunks = maxlen // _ROWS
    in_bufs = scratch[:_NBUF]
    out_bufs = scratch[_NBUF:2 * _NBUF]
    in_sems = scratch[2 * _NBUF]
    out_sems = scratch[2 * _NBUF + 1]

    rows = jax.lax.broadcasted_iota(jnp.int32, (_DOT, _DOT), 0)
    cols = jax.lax.broadcasted_iota(jnp.int32, (_DOT, _DOT), 1)
    perm = (rows + cols == _DOT - 1).astype(jnp.float32)

    def start_in(c):
        src = maxlen - (c + 1) * _ROWS
        pltpu.make_async_copy(
            in_hbm.at[pl.ds(src, _ROWS)], in_bufs[c % _NBUF],
            in_sems.at[c % _NBUF]).start()

    for c in range(min(_NBUF, n_chunks)):
        start_in(c)

    for c in range(n_chunks):
        b = c % _NBUF
        pltpu.make_async_copy(
            in_hbm.at[pl.ds(maxlen - (c + 1) * _ROWS, _ROWS)], in_bufs[b],
            in_sems.at[b]).wait()
        if c >= _NBUF:
            pltpu.make_async_copy(
                out_bufs[b], out_hbm.at[pl.ds((c - _NBUF) * _ROWS, _ROWS)],
                out_sems.at[b]).wait()
        nd = _ROWS // _DOT
        for k in range(nd):
            src = (nd - 1 - k) * _DOT
            out_bufs[b][k * _DOT:(k + 1) * _DOT, :] = jnp.dot(
                perm, in_bufs[b][src:src + _DOT, :],
                preferred_element_type=jnp.float32)
        pltpu.make_async_copy(
            out_bufs[b], out_hbm.at[pl.ds(c * _ROWS, _ROWS)],
            out_sems.at[b]).start()
        nxt = c + _NBUF
        if nxt < n_chunks:
            start_in(nxt)

    for c in range(max(0, n_chunks - _NBUF), n_chunks):
        b = c % _NBUF
        pltpu.make_async_copy(
            out_bufs[b], out_hbm.at[pl.ds(c * _ROWS, _ROWS)],
            out_sems.at[b]).wait()


def kernel(x, pos_emb):
    maxlen = x.shape[-1]
    dim = pos_emb.shape[1]
    return pl.pallas_call(
        _body,
        in_specs=[pl.BlockSpec(memory_space=pl.ANY)],
        out_specs=pl.BlockSpec(memory_space=pl.ANY),
        out_shape=jax.ShapeDtypeStruct((maxlen, dim), pos_emb.dtype),
        scratch_shapes=(
            [pltpu.VMEM((_ROWS, dim), jnp.float32) for _ in range(2 * _NBUF)]
            + [pltpu.SemaphoreType.DMA((_NBUF,)),
               pltpu.SemaphoreType.DMA((_NBUF,))]
        ),
    )(pos_emb[:maxlen])


# speedup vs baseline: 6.8038x; 1.1873x over previous
import jax
import jax.numpy as jnp
from jax.experimental import pallas as pl
from jax.experimental.pallas import tpu as pltpu

_BLOCK = 4096
_CHUNK = 64


def _rev_block(in_ref, out_ref):
    b = in_ref.shape[0]
    rows = jax.lax.broadcasted_iota(jnp.int32, (_CHUNK, _CHUNK), 0)
    cols = jax.lax.broadcasted_iota(jnp.int32, (_CHUNK, _CHUNK), 1)
    perm = (rows + cols == _CHUNK - 1).astype(jnp.float32)
    n = b // _CHUNK
    for k in range(n):
        src = (n - 1 - k) * _CHUNK
        out_ref[k * _CHUNK:(k + 1) * _CHUNK, :] = jnp.dot(
            perm, in_ref[src:src + _CHUNK, :],
            preferred_element_type=jnp.float32)


def kernel(x, pos_emb):
    maxlen = x.shape[-1]
    dim = pos_emb.shape[1]
    num_blocks = maxlen // _BLOCK
    return pl.pallas_call(
        _rev_block,
        grid=(num_blocks,),
        in_specs=[
            pl.BlockSpec((_BLOCK, dim), lambda i: (num_blocks - 1 - i, 0)),
        ],
        out_specs=pl.BlockSpec((_BLOCK, dim), lambda i: (i, 0)),
        out_shape=jax.ShapeDtypeStruct((maxlen, dim), pos_emb.dtype),
        compiler_params=pltpu.CompilerParams(dimension_semantics=('parallel',)),
    )(pos_emb[:maxlen])
